# Initial kernel scaffold; baseline (speedup 1.0000x reference)
import math

import jax
import jax.numpy as jnp
from jax.experimental import pallas as pl
from jax.experimental.pallas import tpu as pltpu

_EPS_BN = 1e-5
_K = 10
_N = 100
_NEG = float("-inf")


def _lrelu(v):
    return jnp.where(v >= 0, v, 0.2 * v)


def _dot(a, b):
    return jnp.dot(a, b, preferred_element_type=jnp.float32)


def _stage(y, a_w, b_w, bias, w2, b2):
    z = _dot(y, a_w)
    base = _dot(y, b_w) + bias - z
    inner = jax.lax.dot_general(
        y, y, (((1,), (1,)), ((), ())), preferred_element_type=jnp.float32
    )
    ii = jax.lax.broadcasted_iota(jnp.int32, (_N, _N), 0)
    jj = jax.lax.broadcasted_iota(jnp.int32, (_N, _N), 1)
    diag = jnp.sum(jnp.where(ii == jj, inner, 0.0), axis=0, keepdims=True)
    pd = 2.0 * inner - diag
    acc = None
    for _ in range(_K):
        m = jnp.max(pd, axis=1, keepdims=True)
        hit = pd >= m
        idx = jnp.min(jnp.where(hit, jj, _N), axis=1, keepdims=True)
        sel = jj == idx
        pd = jnp.where(sel, _NEG, pd)
        t = _dot(sel.astype(jnp.float32), z)
        h = _lrelu(t + base)
        if w2 is not None:
            h = _lrelu(_dot(h, w2) + b2)
        acc = h if acc is None else jnp.maximum(acc, h)
    return acc


def _dgcnn_kernel(
    x_ref,
    w0_ref, b0_ref,
    a1_ref, c1_ref, b1_ref, w2_ref, b2_ref,
    a3_ref, c3_ref, b3_ref, w4_ref, b4_ref,
    a5_ref, c5_ref, b5_ref,
    w6a_ref, w6b_ref, w6c_ref, b6_ref,
    w7g_ref, w7a_ref, w7b_ref, w7c_ref, b7_ref,
    w8_ref, b8_ref, w9_ref,
    out_ref,
):
    x = x_ref[0]
    y0 = _lrelu(_dot(x, w0_ref[:]) + b0_ref[:])
    x1 = _stage(y0, a1_ref[:], c1_ref[:], b1_ref[:], w2_ref[:], b2_ref[:])
    x2 = _stage(x1, a3_ref[:], c3_ref[:], b3_ref[:], w4_ref[:], b4_ref[:])
    x3 = _stage(x2, a5_ref[:], c5_ref[:], b5_ref[:], None, None)
    h6 = _lrelu(
        _dot(x1, w6a_ref[:]) + _dot(x2, w6b_ref[:]) + _dot(x3, w6c_ref[:])
        + b6_ref[:]
    )
    g6 = jnp.max(h6, axis=0, keepdims=True)
    h7 = _lrelu(
        _dot(g6, w7g_ref[:])
        + _dot(x1, w7a_ref[:]) + _dot(x2, w7b_ref[:]) + _dot(x3, w7c_ref[:])
        + b7_ref[:]
    )
    h8 = _lrelu(_dot(h7, w8_ref[:]) + b8_ref[:])
    out_ref[0] = _dot(h8, w9_ref[:])


def kernel(obs, params):
    p = params
    s = 1.0 / math.sqrt(1.0 + _EPS_BN)

    def fold(w, g):
        return (w * (g * s)[:, None]).T

    def bias(name):
        return p["b" + name][None, :]

    w0 = fold(p["W0"], p["g0"])
    w1 = fold(p["W1"], p["g1"])
    w2 = fold(p["W2"], p["g2"])
    w3 = fold(p["W3"], p["g3"])
    w4 = fold(p["W4"], p["g4"])
    w5 = fold(p["W5"], p["g5"])
    w6 = fold(p["W6"], p["g6"])
    w7 = fold(p["W7"], p["g7"])
    w8 = fold(p["W8"], p["g8"])
    w9 = p["W9"].T

    weights = [
        w0, bias("0"),
        w1[:32], w1[32:], bias("1"), w2, bias("2"),
        w3[:32], w3[32:], bias("3"), w4, bias("4"),
        w5[:32], w5[32:], bias("5"),
        w6[:32], w6[32:64], w6[64:], bias("6"),
        w7[:512], w7[512:544], w7[544:576], w7[576:], bias("7"),
        w8, bias("8"), w9,
    ]

    b = obs.shape[0]
    obs3 = obs.reshape(b, _N, obs.shape[1] // _N)

    in_specs = [pl.BlockSpec((1, _N, obs3.shape[2]), lambda i: (i, 0, 0))]
    for w in weights:
        in_specs.append(
            pl.BlockSpec(w.shape, (lambda nd: (lambda i: (0,) * nd))(w.ndim))
        )

    out = pl.pallas_call(
        _dgcnn_kernel,
        grid=(b,),
        in_specs=in_specs,
        out_specs=pl.BlockSpec((1, _N, 1), lambda i: (i, 0, 0)),
        out_shape=jax.ShapeDtypeStruct((b, _N, 1), jnp.float32),
        compiler_params=pltpu.CompilerParams(
            dimension_semantics=("parallel",)
        ),
    )(obs3, *weights)

    q = out.reshape(b, _N)
    return q[None, :, None, :]


# speedup vs baseline: 1.2512x; 1.2512x over previous
import math

import jax
import jax.numpy as jnp
from jax.experimental import pallas as pl
from jax.experimental.pallas import tpu as pltpu

_EPS_BN = 1e-5
_K = 10
_N = 100
_NEG = float("-inf")


def _lrelu(v):
    return jnp.where(v >= 0, v, 0.2 * v)


def _dotd(a, b):
    return jnp.dot(
        a.astype(jnp.bfloat16),
        b.astype(jnp.bfloat16),
        preferred_element_type=jnp.float32,
    )


def _dot_hi(a, b):
    return jax.lax.dot_general(
        a, b, (((1,), (0,)), ((), ())),
        precision=jax.lax.Precision.HIGHEST,
        preferred_element_type=jnp.float32,
    )


def _stage(y, a_w, b_w, gs, bias, w2, gs2, b2):
    yb = y.astype(jnp.bfloat16)
    inner = jax.lax.dot_general(
        yb, yb, (((1,), (1,)), ((), ())), preferred_element_type=jnp.float32
    )
    inner_hi = jax.lax.dot_general(
        y, y, (((1,), (1,)), ((), ())),
        precision=jax.lax.Precision.HIGHEST,
        preferred_element_type=jnp.float32,
    )
    ii = jax.lax.broadcasted_iota(jnp.int32, (_N, _N), 0)
    jj = jax.lax.broadcasted_iota(jnp.int32, (_N, _N), 1)
    diag = jnp.sum(jnp.where(ii == jj, inner_hi, 0.0), axis=0, keepdims=True)
    pd = 2.0 * inner - diag
    base = _dotd(y, b_w)
    acc = None
    for _ in range(_K):
        m = jnp.max(pd, axis=1, keepdims=True)
        hit = pd >= m
        idx = jnp.min(jnp.where(hit, jj, _N), axis=1, keepdims=True)
        sel = jj == idx
        pd = jnp.where(sel, _NEG, pd)
        feat = _dot_hi(sel.astype(jnp.float32), y)
        h = _lrelu((_dotd(feat - y, a_w) + base) * gs + bias)
        if w2 is not None:
            h = _lrelu(_dotd(h, w2) * gs2 + b2)
        acc = h if acc is None else jnp.maximum(acc, h)
    return acc


def _dgcnn_kernel(
    x_ref,
    w0_ref, g0_ref, b0_ref,
    a1_ref, c1_ref, g1_ref, b1_ref, w2_ref, g2_ref, b2_ref,
    a3_ref, c3_ref, g3_ref, b3_ref, w4_ref, g4_ref, b4_ref,
    a5_ref, c5_ref, g5_ref, b5_ref,
    w6a_ref, w6b_ref, w6c_ref, g6_ref, b6_ref,
    w7g_ref, w7a_ref, w7b_ref, w7c_ref, g7_ref, b7_ref,
    w8_ref, g8_ref, b8_ref, w9_ref,
    out_ref,
):
    x = x_ref[0]
    y0 = _lrelu(_dotd(x, w0_ref[:]) * g0_ref[:] + b0_ref[:])
    x1 = _stage(y0, a1_ref[:], c1_ref[:], g1_ref[:], b1_ref[:],
                w2_ref[:], g2_ref[:], b2_ref[:])
    x2 = _stage(x1, a3_ref[:], c3_ref[:], g3_ref[:], b3_ref[:],
                w4_ref[:], g4_ref[:], b4_ref[:])
    x3 = _stage(x2, a5_ref[:], c5_ref[:], g5_ref[:], b5_ref[:],
                None, None, None)
    h6 = _lrelu(
        (_dotd(x1, w6a_ref[:]) + _dotd(x2, w6b_ref[:]) + _dotd(x3, w6c_ref[:]))
        * g6_ref[:] + b6_ref[:]
    )
    g6 = jnp.max(h6, axis=0, keepdims=True)
    h7 = _lrelu(
        (_dotd(g6, w7g_ref[:])
         + _dotd(x1, w7a_ref[:]) + _dotd(x2, w7b_ref[:])
         + _dotd(x3, w7c_ref[:]))
        * g7_ref[:] + b7_ref[:]
    )
    h8 = _lrelu(_dotd(h7, w8_ref[:]) * g8_ref[:] + b8_ref[:])
    out_ref[0] = _dotd(h8, w9_ref[:])


def kernel(obs, params):
    p = params
    s = 1.0 / math.sqrt(1.0 + _EPS_BN)

    def gs(name):
        return (p["g" + name] * s)[None, :]

    def bias(name):
        return p["b" + name][None, :]

    w0 = p["W0"].T
    w1 = p["W1"].T
    w2 = p["W2"].T
    w3 = p["W3"].T
    w4 = p["W4"].T
    w5 = p["W5"].T
    w6 = p["W6"].T
    w7 = p["W7"].T
    w8 = p["W8"].T
    w9 = p["W9"].T

    weights = [
        w0, gs("0"), bias("0"),
        w1[:32], w1[32:], gs("1"), bias("1"), w2, gs("2"), bias("2"),
        w3[:32], w3[32:], gs("3"), bias("3"), w4, gs("4"), bias("4"),
        w5[:32], w5[32:], gs("5"), bias("5"),
        w6[:32], w6[32:64], w6[64:], gs("6"), bias("6"),
        w7[:512], w7[512:544], w7[544:576], w7[576:], gs("7"), bias("7"),
        w8, gs("8"), bias("8"), w9,
    ]

    b = obs.shape[0]
    obs3 = obs.reshape(b, _N, obs.shape[1] // _N)

    in_specs = [pl.BlockSpec((1, _N, obs3.shape[2]), lambda i: (i, 0, 0))]
    for w in weights:
        in_specs.append(
            pl.BlockSpec(w.shape, (lambda nd: (lambda i: (0,) * nd))(w.ndim))
        )

    out = pl.pallas_call(
        _dgcnn_kernel,
        grid=(b,),
        in_specs=in_specs,
        out_specs=pl.BlockSpec((1, _N, 1), lambda i: (i, 0, 0)),
        out_shape=jax.ShapeDtypeStruct((b, _N, 1), jnp.float32),
        compiler_params=pltpu.CompilerParams(
            dimension_semantics=("parallel",)
        ),
    )(obs3, *weights)

    q = out.reshape(b, _N)
    return q[None, :, None, :]


# speedup vs baseline: 1.2723x; 1.0168x over previous
import math

import jax
import jax.numpy as jnp
from jax.experimental import pallas as pl
from jax.experimental.pallas import tpu as pltpu

_EPS_BN = 1e-5
_K = 10
_N = 100
_NEG = float("-inf")


def _lrelu(v):
    return jnp.where(v >= 0, v, 0.2 * v)


def _dotd(a, b):
    return jnp.dot(
        a.astype(jnp.bfloat16),
        b.astype(jnp.bfloat16),
        preferred_element_type=jnp.float32,
    )


def _dot_hi(a, b):
    return jax.lax.dot_general(
        a, b, (((1,), (0,)), ((), ())),
        precision=jax.lax.Precision.HIGHEST,
        preferred_element_type=jnp.float32,
    )


def _stage(y, a_w, b_w, gs, bias, w2, gs2, b2):
    yb = y.astype(jnp.bfloat16)
    inner = jax.lax.dot_general(
        yb, yb, (((1,), (1,)), ((), ())), preferred_element_type=jnp.float32
    )
    norms = jnp.transpose(jnp.sum(y * y, axis=1, keepdims=True))
    jj = jax.lax.broadcasted_iota(jnp.int32, (_N, _N), 1)
    pd = 2.0 * inner - norms
    sels = []
    for _ in range(_K):
        m = jnp.max(pd, axis=1, keepdims=True)
        hit = pd >= m
        idx = jnp.min(jnp.where(hit, jj, _N), axis=1, keepdims=True)
        sel = jj == idx
        pd = jnp.where(sel, _NEG, pd)
        sels.append(sel.astype(jnp.float32))
    msel = jnp.concatenate(sels, axis=0)
    feat = _dot_hi(msel, y)
    d = feat.reshape(_K, _N, 32) - y[None]
    base = _dotd(y, b_w)
    t = _dotd(d.reshape(_K * _N, 32), a_w).reshape(_K, _N, 32)
    h = _lrelu((t + base[None]) * gs + bias)
    if w2 is not None:
        t2 = _dotd(h.reshape(_K * _N, 32), w2).reshape(_K, _N, 32)
        h = _lrelu(t2 * gs2 + b2)
    return jnp.max(h, axis=0)


def _dgcnn_kernel(
    x_ref,
    w0_ref, g0_ref, b0_ref,
    a1_ref, c1_ref, g1_ref, b1_ref, w2_ref, g2_ref, b2_ref,
    a3_ref, c3_ref, g3_ref, b3_ref, w4_ref, g4_ref, b4_ref,
    a5_ref, c5_ref, g5_ref, b5_ref,
    w6a_ref, w6b_ref, w6c_ref, g6_ref, b6_ref,
    w7g_ref, w7a_ref, w7b_ref, w7c_ref, g7_ref, b7_ref,
    w8_ref, g8_ref, b8_ref, w9_ref,
    out_ref,
):
    x = x_ref[0]
    y0 = _lrelu(_dotd(x, w0_ref[:]) * g0_ref[:] + b0_ref[:])
    x1 = _stage(y0, a1_ref[:], c1_ref[:], g1_ref[:], b1_ref[:],
                w2_ref[:], g2_ref[:], b2_ref[:])
    x2 = _stage(x1, a3_ref[:], c3_ref[:], g3_ref[:], b3_ref[:],
                w4_ref[:], g4_ref[:], b4_ref[:])
    x3 = _stage(x2, a5_ref[:], c5_ref[:], g5_ref[:], b5_ref[:],
                None, None, None)
    h6 = _lrelu(
        (_dotd(x1, w6a_ref[:]) + _dotd(x2, w6b_ref[:]) + _dotd(x3, w6c_ref[:]))
        * g6_ref[:] + b6_ref[:]
    )
    g6 = jnp.max(h6, axis=0, keepdims=True)
    h7 = _lrelu(
        (_dotd(g6, w7g_ref[:])
         + _dotd(x1, w7a_ref[:]) + _dotd(x2, w7b_ref[:])
         + _dotd(x3, w7c_ref[:]))
        * g7_ref[:] + b7_ref[:]
    )
    h8 = _lrelu(_dotd(h7, w8_ref[:]) * g8_ref[:] + b8_ref[:])
    out_ref[0] = _dotd(h8, w9_ref[:])


def kernel(obs, params):
    p = params
    s = 1.0 / math.sqrt(1.0 + _EPS_BN)

    def gs(name):
        return (p["g" + name] * s)[None, :]

    def bias(name):
        return p["b" + name][None, :]

    w0 = p["W0"].T
    w1 = p["W1"].T
    w2 = p["W2"].T
    w3 = p["W3"].T
    w4 = p["W4"].T
    w5 = p["W5"].T
    w6 = p["W6"].T
    w7 = p["W7"].T
    w8 = p["W8"].T
    w9 = p["W9"].T

    weights = [
        w0, gs("0"), bias("0"),
        w1[:32], w1[32:], gs("1"), bias("1"), w2, gs("2"), bias("2"),
        w3[:32], w3[32:], gs("3"), bias("3"), w4, gs("4"), bias("4"),
        w5[:32], w5[32:], gs("5"), bias("5"),
        w6[:32], w6[32:64], w6[64:], gs("6"), bias("6"),
        w7[:512], w7[512:544], w7[544:576], w7[576:], gs("7"), bias("7"),
        w8, gs("8"), bias("8"), w9,
    ]

    b = obs.shape[0]
    obs3 = obs.reshape(b, _N, obs.shape[1] // _N)

    in_specs = [pl.BlockSpec((1, _N, obs3.shape[2]), lambda i: (i, 0, 0))]
    for w in weights:
        in_specs.append(
            pl.BlockSpec(w.shape, (lambda nd: (lambda i: (0,) * nd))(w.ndim))
        )

    out = pl.pallas_call(
        _dgcnn_kernel,
        grid=(b,),
        in_specs=in_specs,
        out_specs=pl.BlockSpec((1, _N, 1), lambda i: (i, 0, 0)),
        out_shape=jax.ShapeDtypeStruct((b, _N, 1), jnp.float32),
        compiler_params=pltpu.CompilerParams(
            dimension_semantics=("parallel",)
        ),
    )(obs3, *weights)

    q = out.reshape(b, _N)
    return q[None, :, None, :]
